# baseline (speedup 1.0000x reference)
import functools

import jax
import jax.numpy as jnp
from jax import lax
from jax.experimental import pallas as pl
from jax.experimental.pallas import tpu as pltpu
from jax.experimental.pallas import tpu_sc as plsc

N = 20000
MAX_DET = 100
IOU_THRESH = 0.5
SCORE_THRESH = 0.05
IMG_SIZE = 416.0

NUM_TILES = 16
LANES = 16
PAD_N = 20480
PER_TILE = PAD_N // NUM_TILES
CHUNKS = PER_TILE // LANES
SZ = PER_TILE + LANES
NEG = -1.0e30
NEG_TEST = -1.0e29
BIG = 3.0e38
COMPACT_EVERY = 16


def _lanes_f32():
    return lax.iota(jnp.int32, LANES).astype(jnp.float32)


def _build_record(vals):
    li = lax.iota(jnp.int32, LANES)
    rec = jnp.zeros((LANES,), jnp.float32)
    for k, v in enumerate(vals):
        rec = jnp.where(li == k, v, rec)
    return rec


def _nms_body(cx_hbm, cy_hbm, w_hbm, h_hbm, s_hbm, out_hbm,
              x1, y1, x2, y2, area, sc, recbuf, rec_all, outbuf, shared):
    wid = lax.axis_index("s")
    base = wid * PER_TILE

    pltpu.sync_copy(cx_hbm.at[pl.ds(base, PER_TILE)], x1.at[pl.ds(0, PER_TILE)])
    pltpu.sync_copy(w_hbm.at[pl.ds(base, PER_TILE)], x2.at[pl.ds(0, PER_TILE)])
    pltpu.sync_copy(cy_hbm.at[pl.ds(base, PER_TILE)], y1.at[pl.ds(0, PER_TILE)])
    pltpu.sync_copy(h_hbm.at[pl.ds(base, PER_TILE)], y2.at[pl.ds(0, PER_TILE)])
    pltpu.sync_copy(s_hbm.at[pl.ds(base, PER_TILE)], sc.at[pl.ds(0, PER_TILE)])

    lanes = _lanes_f32()
    li = lax.iota(jnp.int32, LANES)

    def decode_chunk(c, carry):
        bv, bi = carry
        d = pl.ds(c * LANES, LANES)
        cxv = x1[d] * IMG_SIZE
        wv = x2[d] * IMG_SIZE
        cyv = y1[d] * IMG_SIZE
        hv = y2[d] * IMG_SIZE
        x1v = cxv - wv * 0.5
        x2v = cxv + wv * 0.5
        y1v = cyv - hv * 0.5
        y2v = cyv + hv * 0.5
        x1[d] = x1v
        x2[d] = x2v
        y1[d] = y1v
        y2[d] = y2v
        area[d] = jnp.maximum(x2v - x1v, 0.0) * jnp.maximum(y2v - y1v, 0.0)
        sv = sc[d]
        sv = jnp.where(sv > SCORE_THRESH, sv, NEG)
        sc[d] = sv
        posv = (c * LANES).astype(jnp.float32) + lanes
        upd = sv > bv
        return jnp.where(upd, sv, bv), jnp.where(upd, posv, bi)

    bv0 = jnp.full((LANES,), -BIG, jnp.float32)
    bi0 = jnp.zeros((LANES,), jnp.float32)
    bv, bi = lax.fori_loop(0, CHUNKS, decode_chunk, (bv0, bi0))

    def step(d, carry):
        bv, bi, nlive = carry
        lval = jnp.max(bv)
        lpos = jnp.min(jnp.where(bv == lval, bi, BIG))
        off = lpos.astype(jnp.int32)
        cbase = (off // LANES) * LANES
        lane = (off - cbase).astype(jnp.float32)
        dsl = pl.ds(cbase, LANES)
        lm = lanes == lane
        gx1 = jnp.max(jnp.where(lm, x1[dsl], -BIG))
        gy1 = jnp.max(jnp.where(lm, y1[dsl], -BIG))
        gx2 = jnp.max(jnp.where(lm, x2[dsl], -BIG))
        gy2 = jnp.max(jnp.where(lm, y2[dsl], -BIG))
        gar = jnp.max(jnp.where(lm, area[dsl], -BIG))
        recbuf[...] = _build_record(
            [lval, wid.astype(jnp.float32), gx1, gy1, gx2, gy2, gar])
        pbase = (d % 2) * (NUM_TILES * LANES)
        pltpu.sync_copy(recbuf, shared.at[pl.ds(pbase + wid * LANES, LANES)])
        plsc.subcore_barrier()
        pltpu.sync_copy(shared.at[pl.ds(pbase, NUM_TILES * LANES)], rec_all)

        def _lane0(vec):
            return jnp.max(jnp.where(li == 0, vec, -BIG))

        wrec = rec_all[pl.ds(0, LANES)]
        wv_ = _lane0(wrec)
        for j in range(1, NUM_TILES):
            rj = rec_all[pl.ds(j * LANES, LANES)]
            vj = _lane0(rj)
            pf = jnp.where(vj > wv_, 1.0, 0.0)
            pv = jnp.zeros((LANES,), jnp.float32) + pf
            wrec = wrec * (1.0 - pv) + rj * pv
            wv_ = jnp.maximum(wv_, vj)

        def _lane(k):
            return jnp.max(jnp.where(li == k, wrec, -BIG))

        wtile = _lane(1)
        wx1 = _lane(2)
        wy1 = _lane(3)
        wx2 = _lane(4)
        wy2 = _lane(5)
        war = _lane(6)

        @pl.when(wid == 0)
        def _():
            valid = wv_ > 0.0
            z = jnp.float32(0.0)
            outbuf[d] = _build_record([
                jnp.where(valid, wx1, z),
                jnp.where(valid, wy1, z),
                jnp.where(valid, wx2, z),
                jnp.where(valid, wy2, z),
                jnp.where(valid, wv_, z),
            ])


        def _compact():
            nch_o = (nlive + (LANES - 1)) // LANES

            def cloop(c, pos):
                dd = pl.ds(c * LANES, LANES)
                sv = sc[dd]
                m = sv > NEG_TEST
                plsc.store_compressed(x1.at[pl.ds(pos, LANES)], x1[dd], mask=m)
                plsc.store_compressed(y1.at[pl.ds(pos, LANES)], y1[dd], mask=m)
                plsc.store_compressed(x2.at[pl.ds(pos, LANES)], x2[dd], mask=m)
                plsc.store_compressed(y2.at[pl.ds(pos, LANES)], y2[dd], mask=m)
                plsc.store_compressed(area.at[pl.ds(pos, LANES)], area[dd],
                                      mask=m)
                plsc.store_compressed(sc.at[pl.ds(pos, LANES)], sv, mask=m)
                n = plsc.all_reduce_population_count(m)
                return pos + jnp.max(n)

            newn = lax.fori_loop(0, nch_o, cloop, jnp.int32(0))
            tb = (newn // LANES) * LANES
            tl = newn - tb
            sc[pl.ds(tb, LANES)] = jnp.where(li >= tl, NEG, sc[pl.ds(tb, LANES)])
            return newn

        nlive2 = nlive

        own = wid.astype(jnp.float32) == wtile
        wpos = jnp.where(own, lpos, -1.0)

        def sweep(c, carry):
            bv, bi = carry
            dd = pl.ds(c * LANES, LANES)
            x1v = x1[dd]
            y1v = y1[dd]
            x2v = x2[dd]
            y2v = y2[dd]
            ix1 = jnp.maximum(wx1, x1v)
            iy1 = jnp.maximum(wy1, y1v)
            ix2 = jnp.minimum(wx2, x2v)
            iy2 = jnp.minimum(wy2, y2v)
            inter = jnp.maximum(ix2 - ix1, 0.0) * jnp.maximum(iy2 - iy1, 0.0)
            union = war + area[dd] - inter
            iou = inter / jnp.maximum(union, 1e-9)
            posv = (c * LANES).astype(jnp.float32) + lanes
            supp = (iou > IOU_THRESH) | ((posv == wpos) & own)
            nv = jnp.where(supp, NEG, sc[dd])
            sc[dd] = nv
            upd = nv > bv
            return jnp.where(upd, nv, bv), jnp.where(upd, posv, bi)

        bvn = jnp.full((LANES,), -BIG, jnp.float32)
        bin_ = jnp.zeros((LANES,), jnp.float32)
        bvn, bin_ = lax.fori_loop(0, CHUNKS, sweep, (bvn, bin_))
        return bvn, bin_, nlive2

    lax.fori_loop(0, MAX_DET, step, (bv, bi, jnp.int32(PER_TILE)))

    @pl.when(wid == 0)
    def _():
        pltpu.sync_copy(outbuf, out_hbm)


@jax.jit
def _nms_sc(cx, cy, w, h, s):
    mesh = plsc.VectorSubcoreMesh(
        core_axis_name="c", subcore_axis_name="s",
        num_cores=1, num_subcores=NUM_TILES)
    f = functools.partial(
        pl.kernel,
        out_type=jax.ShapeDtypeStruct((MAX_DET, LANES), jnp.float32),
        mesh=mesh,
        compiler_params=pltpu.CompilerParams(needs_layout_passes=False),
        scratch_types=[
            pltpu.VMEM((SZ,), jnp.float32),
            pltpu.VMEM((SZ,), jnp.float32),
            pltpu.VMEM((SZ,), jnp.float32),
            pltpu.VMEM((SZ,), jnp.float32),
            pltpu.VMEM((SZ,), jnp.float32),
            pltpu.VMEM((SZ,), jnp.float32),
            pltpu.VMEM((LANES,), jnp.float32),
            pltpu.VMEM((NUM_TILES * LANES,), jnp.float32),
            pltpu.VMEM((MAX_DET, LANES), jnp.float32),
            pltpu.VMEM_SHARED((2 * NUM_TILES * LANES,), jnp.float32),
        ],
    )(_nms_body)
    return f(cx, cy, w, h, s)


def kernel(boxes, scores):
    pad = PAD_N - N
    cx = jnp.pad(boxes[:, 0], (0, pad))
    cy = jnp.pad(boxes[:, 1], (0, pad))
    w = jnp.pad(boxes[:, 2], (0, pad))
    h = jnp.pad(boxes[:, 3], (0, pad))
    s = jnp.pad(scores, (0, pad))
    out = _nms_sc(cx, cy, w, h, s)
    return out[:, :5]


# speedup vs baseline: 2.1690x; 2.1690x over previous
import functools

import jax
import jax.numpy as jnp
from jax import lax
from jax.experimental import pallas as pl
from jax.experimental.pallas import tpu as pltpu
from jax.experimental.pallas import tpu_sc as plsc

N = 20000
MAX_DET = 100
IOU_THRESH = 0.5
SCORE_THRESH = 0.05
IMG_SIZE = 416.0

NUM_TILES = 16
LANES = 16
PAD_N = 20480
PER_TILE = PAD_N // NUM_TILES
CHUNKS = PER_TILE // LANES
NEG = -1.0e30
BIG = 3.0e38


def _lanes_f32():
    return lax.iota(jnp.int32, LANES).astype(jnp.float32)


def _build_record(vals):
    li = lax.iota(jnp.int32, LANES)
    rec = jnp.zeros((LANES,), jnp.float32)
    for k, v in enumerate(vals):
        rec = jnp.where(li == k, v, rec)
    return rec


def _nms_body(cx_hbm, cy_hbm, w_hbm, h_hbm, s_hbm, out_hbm,
              x1, y1, x2, y2, area, sc, recbuf, rec_all, outbuf, shared):
    wid = lax.axis_index("s")
    base = wid * PER_TILE

    pltpu.sync_copy(cx_hbm.at[pl.ds(base, PER_TILE)], x1)
    pltpu.sync_copy(w_hbm.at[pl.ds(base, PER_TILE)], x2)
    pltpu.sync_copy(cy_hbm.at[pl.ds(base, PER_TILE)], y1)
    pltpu.sync_copy(h_hbm.at[pl.ds(base, PER_TILE)], y2)
    pltpu.sync_copy(s_hbm.at[pl.ds(base, PER_TILE)], sc)

    lanes = _lanes_f32()

    def decode_chunk(c, carry):
        bv, bi = carry
        d = pl.ds(c * LANES, LANES)
        cxv = x1[d] * IMG_SIZE
        wv = x2[d] * IMG_SIZE
        cyv = y1[d] * IMG_SIZE
        hv = y2[d] * IMG_SIZE
        x1v = cxv - wv * 0.5
        x2v = cxv + wv * 0.5
        y1v = cyv - hv * 0.5
        y2v = cyv + hv * 0.5
        x1[d] = x1v
        x2[d] = x2v
        y1[d] = y1v
        y2[d] = y2v
        area[d] = jnp.maximum(x2v - x1v, 0.0) * jnp.maximum(y2v - y1v, 0.0)
        sv = sc[d]
        sv = jnp.where(sv > SCORE_THRESH, sv, NEG)
        sc[d] = sv
        idxv = (base + c * LANES).astype(jnp.float32) + lanes
        upd = sv > bv
        return jnp.where(upd, sv, bv), jnp.where(upd, idxv, bi)

    bv0 = jnp.full((LANES,), -BIG, jnp.float32)
    bi0 = jnp.zeros((LANES,), jnp.float32)
    bv, bi = lax.fori_loop(0, CHUNKS, decode_chunk, (bv0, bi0))

    def step(d, carry):
        bv, bi = carry
        lval = jnp.max(bv)
        lidx = jnp.min(jnp.where(bv == lval, bi, BIG))
        off = lidx.astype(jnp.int32) - base
        cbase = (off // LANES) * LANES
        lane = (off - cbase).astype(jnp.float32)
        dsl = pl.ds(cbase, LANES)
        lm = lanes == lane
        gx1 = jnp.max(jnp.where(lm, x1[dsl], -BIG))
        gy1 = jnp.max(jnp.where(lm, y1[dsl], -BIG))
        gx2 = jnp.max(jnp.where(lm, x2[dsl], -BIG))
        gy2 = jnp.max(jnp.where(lm, y2[dsl], -BIG))
        gar = jnp.max(jnp.where(lm, area[dsl], -BIG))
        recbuf[...] = _build_record([lval, lidx, gx1, gy1, gx2, gy2, gar])
        pbase = (d % 2) * (NUM_TILES * LANES)
        pltpu.sync_copy(recbuf, shared.at[pl.ds(pbase + wid * LANES, LANES)])
        plsc.subcore_barrier()
        pltpu.sync_copy(shared.at[pl.ds(pbase, NUM_TILES * LANES)], rec_all)

        li = lax.iota(jnp.int32, LANES)

        def _lane0(vec):
            return jnp.max(jnp.where(li == 0, vec, -BIG))

        wrec = rec_all[pl.ds(0, LANES)]
        wv_ = _lane0(wrec)
        for j in range(1, NUM_TILES):
            rj = rec_all[pl.ds(j * LANES, LANES)]
            vj = _lane0(rj)
            pf = jnp.where(vj > wv_, 1.0, 0.0)
            pv = jnp.zeros((LANES,), jnp.float32) + pf
            wrec = wrec * (1.0 - pv) + rj * pv
            wv_ = jnp.maximum(wv_, vj)

        def _lane(k):
            return jnp.max(jnp.where(li == k, wrec, -BIG))

        wi_ = _lane(1)
        wx1 = _lane(2)
        wy1 = _lane(3)
        wx2 = _lane(4)
        wy2 = _lane(5)
        war = _lane(6)

        @pl.when(wid == 0)
        def _():
            valid = wv_ > 0.0
            z = jnp.float32(0.0)
            outbuf[d] = _build_record([
                jnp.where(valid, wx1, z),
                jnp.where(valid, wy1, z),
                jnp.where(valid, wx2, z),
                jnp.where(valid, wy2, z),
                jnp.where(valid, wv_, z),
            ])

        def sweep(c, carry):
            bv, bi = carry
            dd = pl.ds(c * LANES, LANES)
            x1v = x1[dd]
            y1v = y1[dd]
            x2v = x2[dd]
            y2v = y2[dd]
            ix1 = jnp.maximum(wx1, x1v)
            iy1 = jnp.maximum(wy1, y1v)
            ix2 = jnp.minimum(wx2, x2v)
            iy2 = jnp.minimum(wy2, y2v)
            inter = jnp.maximum(ix2 - ix1, 0.0) * jnp.maximum(iy2 - iy1, 0.0)
            union = war + area[dd] - inter
            iou = inter / jnp.maximum(union, 1e-9)
            idxv = (base + c * LANES).astype(jnp.float32) + lanes
            supp = (iou > IOU_THRESH) | (idxv == wi_)
            nv = jnp.where(supp, NEG, sc[dd])
            sc[dd] = nv
            upd = nv > bv
            return jnp.where(upd, nv, bv), jnp.where(upd, idxv, bi)

        bvn = jnp.full((LANES,), -BIG, jnp.float32)
        bin_ = jnp.zeros((LANES,), jnp.float32)
        return lax.fori_loop(0, CHUNKS, sweep, (bvn, bin_))

    lax.fori_loop(0, MAX_DET, step, (bv, bi))

    @pl.when(wid == 0)
    def _():
        pltpu.sync_copy(outbuf, out_hbm)


@jax.jit
def _nms_sc(cx, cy, w, h, s):
    mesh = plsc.VectorSubcoreMesh(
        core_axis_name="c", subcore_axis_name="s",
        num_cores=1, num_subcores=NUM_TILES)
    f = functools.partial(
        pl.kernel,
        out_type=jax.ShapeDtypeStruct((MAX_DET, LANES), jnp.float32),
        mesh=mesh,
        compiler_params=pltpu.CompilerParams(needs_layout_passes=False),
        scratch_types=[
            pltpu.VMEM((PER_TILE,), jnp.float32),
            pltpu.VMEM((PER_TILE,), jnp.float32),
            pltpu.VMEM((PER_TILE,), jnp.float32),
            pltpu.VMEM((PER_TILE,), jnp.float32),
            pltpu.VMEM((PER_TILE,), jnp.float32),
            pltpu.VMEM((PER_TILE,), jnp.float32),
            pltpu.VMEM((LANES,), jnp.float32),
            pltpu.VMEM((NUM_TILES * LANES,), jnp.float32),
            pltpu.VMEM((MAX_DET, LANES), jnp.float32),
            pltpu.VMEM_SHARED((2 * NUM_TILES * LANES,), jnp.float32),
        ],
    )(_nms_body)
    return f(cx, cy, w, h, s)


def kernel(boxes, scores):
    pad = PAD_N - N
    cx = jnp.pad(boxes[:, 0], (0, pad))
    cy = jnp.pad(boxes[:, 1], (0, pad))
    w = jnp.pad(boxes[:, 2], (0, pad))
    h = jnp.pad(boxes[:, 3], (0, pad))
    s = jnp.pad(scores, (0, pad))
    out = _nms_sc(cx, cy, w, h, s)
    return out[:, :5]


# speedup vs baseline: 2.2466x; 1.0358x over previous
import functools

import jax
import jax.numpy as jnp
from jax import lax
from jax.experimental import pallas as pl
from jax.experimental.pallas import tpu as pltpu
from jax.experimental.pallas import tpu_sc as plsc

N = 20000
MAX_DET = 100
IOU_THRESH = 0.5
SCORE_THRESH = 0.05
IMG_SIZE = 416.0

NUM_TILES = 16
LANES = 16
PAD_N = 20480
PER_TILE = PAD_N // NUM_TILES
CHUNKS = PER_TILE // LANES
NEG = -1.0e30
BIG = 3.0e38


def _lanes_f32():
    return lax.iota(jnp.int32, LANES).astype(jnp.float32)


def _build_record(vals):
    li = lax.iota(jnp.int32, LANES)
    rec = jnp.zeros((LANES,), jnp.float32)
    for k, v in enumerate(vals):
        rec = jnp.where(li == k, v, rec)
    return rec


def _nms_body(cx_hbm, cy_hbm, w_hbm, h_hbm, s_hbm, out_hbm,
              x1, y1, x2, y2, area, sc, recbuf, rec_all, outbuf, shared):
    wid = lax.axis_index("s")
    base = wid * PER_TILE

    pltpu.sync_copy(cx_hbm.at[pl.ds(base, PER_TILE)], x1)
    pltpu.sync_copy(w_hbm.at[pl.ds(base, PER_TILE)], x2)
    pltpu.sync_copy(cy_hbm.at[pl.ds(base, PER_TILE)], y1)
    pltpu.sync_copy(h_hbm.at[pl.ds(base, PER_TILE)], y2)
    pltpu.sync_copy(s_hbm.at[pl.ds(base, PER_TILE)], sc)

    lanes = _lanes_f32()

    def decode_chunk(c, carry):
        bv, bi = carry
        d = pl.ds(c * LANES, LANES)
        cxv = x1[d] * IMG_SIZE
        wv = x2[d] * IMG_SIZE
        cyv = y1[d] * IMG_SIZE
        hv = y2[d] * IMG_SIZE
        x1v = cxv - wv * 0.5
        x2v = cxv + wv * 0.5
        y1v = cyv - hv * 0.5
        y2v = cyv + hv * 0.5
        x1[d] = x1v
        x2[d] = x2v
        y1[d] = y1v
        y2[d] = y2v
        area[d] = jnp.maximum(x2v - x1v, 0.0) * jnp.maximum(y2v - y1v, 0.0)
        sv = sc[d]
        sv = jnp.where(sv > SCORE_THRESH, sv, NEG)
        sc[d] = sv
        idxv = (base + c * LANES).astype(jnp.float32) + lanes
        upd = sv > bv
        return jnp.where(upd, sv, bv), jnp.where(upd, idxv, bi)

    bv0 = jnp.full((LANES,), -BIG, jnp.float32)
    bi0 = jnp.zeros((LANES,), jnp.float32)
    bv, bi = lax.fori_loop(0, CHUNKS, decode_chunk, (bv0, bi0))

    def step(d, carry):
        bv, bi = carry
        lval = jnp.max(bv)
        lidx = jnp.min(jnp.where(bv == lval, bi, BIG))
        off = lidx.astype(jnp.int32) - base
        offv = jnp.zeros((LANES,), jnp.int32) + off
        gx1 = plsc.load_gather(x1, [offv])
        gy1 = plsc.load_gather(y1, [offv])
        gx2 = plsc.load_gather(x2, [offv])
        gy2 = plsc.load_gather(y2, [offv])
        gar = plsc.load_gather(area, [offv])
        recbuf[...] = _build_record([lval, lidx, gx1, gy1, gx2, gy2, gar])
        pbase = (d % 2) * (NUM_TILES * LANES)
        pltpu.sync_copy(recbuf, shared.at[pl.ds(pbase + wid * LANES, LANES)])
        plsc.subcore_barrier()
        pltpu.sync_copy(shared.at[pl.ds(pbase, NUM_TILES * LANES)], rec_all)

        li = lax.iota(jnp.int32, LANES)

        vals = plsc.load_gather(rec_all, [li * LANES])
        wv_ = jnp.max(vals)
        wtf = jnp.min(jnp.where(vals == wv_, lanes, BIG))
        wrec = rec_all[pl.ds(wtf.astype(jnp.int32) * LANES, LANES)]

        def _lane(k):
            return jnp.max(jnp.where(li == k, wrec, -BIG))

        wi_ = _lane(1)
        wx1 = _lane(2)
        wy1 = _lane(3)
        wx2 = _lane(4)
        wy2 = _lane(5)
        war = _lane(6)

        @pl.when(wid == 0)
        def _():
            valid = wv_ > 0.0
            z = jnp.float32(0.0)
            outbuf[d] = _build_record([
                jnp.where(valid, wx1, z),
                jnp.where(valid, wy1, z),
                jnp.where(valid, wx2, z),
                jnp.where(valid, wy2, z),
                jnp.where(valid, wv_, z),
            ])

        def sweep(c, carry):
            bv, bi = carry
            dd = pl.ds(c * LANES, LANES)
            x1v = x1[dd]
            y1v = y1[dd]
            x2v = x2[dd]
            y2v = y2[dd]
            ix1 = jnp.maximum(wx1, x1v)
            iy1 = jnp.maximum(wy1, y1v)
            ix2 = jnp.minimum(wx2, x2v)
            iy2 = jnp.minimum(wy2, y2v)
            inter = jnp.maximum(ix2 - ix1, 0.0) * jnp.maximum(iy2 - iy1, 0.0)
            union = war + area[dd] - inter
            iou = inter / jnp.maximum(union, 1e-9)
            idxv = (base + c * LANES).astype(jnp.float32) + lanes
            supp = (iou > IOU_THRESH) | (idxv == wi_)
            nv = jnp.where(supp, NEG, sc[dd])
            sc[dd] = nv
            upd = nv > bv
            return jnp.where(upd, nv, bv), jnp.where(upd, idxv, bi)

        bvn = jnp.full((LANES,), -BIG, jnp.float32)
        bin_ = jnp.zeros((LANES,), jnp.float32)
        return lax.fori_loop(0, CHUNKS, sweep, (bvn, bin_))

    lax.fori_loop(0, MAX_DET, step, (bv, bi))

    @pl.when(wid == 0)
    def _():
        pltpu.sync_copy(outbuf, out_hbm)


@jax.jit
def _nms_sc(cx, cy, w, h, s):
    mesh = plsc.VectorSubcoreMesh(
        core_axis_name="c", subcore_axis_name="s",
        num_cores=1, num_subcores=NUM_TILES)
    f = functools.partial(
        pl.kernel,
        out_type=jax.ShapeDtypeStruct((MAX_DET, LANES), jnp.float32),
        mesh=mesh,
        compiler_params=pltpu.CompilerParams(needs_layout_passes=False),
        scratch_types=[
            pltpu.VMEM((PER_TILE,), jnp.float32),
            pltpu.VMEM((PER_TILE,), jnp.float32),
            pltpu.VMEM((PER_TILE,), jnp.float32),
            pltpu.VMEM((PER_TILE,), jnp.float32),
            pltpu.VMEM((PER_TILE,), jnp.float32),
            pltpu.VMEM((PER_TILE,), jnp.float32),
            pltpu.VMEM((LANES,), jnp.float32),
            pltpu.VMEM((NUM_TILES * LANES,), jnp.float32),
            pltpu.VMEM((MAX_DET, LANES), jnp.float32),
            pltpu.VMEM_SHARED((2 * NUM_TILES * LANES,), jnp.float32),
        ],
    )(_nms_body)
    return f(cx, cy, w, h, s)


def kernel(boxes, scores):
    pad = PAD_N - N
    cx = jnp.pad(boxes[:, 0], (0, pad))
    cy = jnp.pad(boxes[:, 1], (0, pad))
    w = jnp.pad(boxes[:, 2], (0, pad))
    h = jnp.pad(boxes[:, 3], (0, pad))
    s = jnp.pad(scores, (0, pad))
    out = _nms_sc(cx, cy, w, h, s)
    return out[:, :5]


# speedup vs baseline: 2.2909x; 1.0197x over previous
import functools

import jax
import jax.numpy as jnp
from jax import lax
from jax.experimental import pallas as pl
from jax.experimental.pallas import tpu as pltpu
from jax.experimental.pallas import tpu_sc as plsc

N = 20000
MAX_DET = 100
IOU_THRESH = 0.5
SCORE_THRESH = 0.05
IMG_SIZE = 416.0

NUM_TILES = 16
LANES = 16
PAD_N = 20480
PER_TILE = PAD_N // NUM_TILES
CHUNKS = PER_TILE // LANES
NEG = -1.0e30
BIG = 3.0e38


def _lanes_f32():
    return lax.iota(jnp.int32, LANES).astype(jnp.float32)


def _build_record(vals):
    li = lax.iota(jnp.int32, LANES)
    rec = jnp.zeros((LANES,), jnp.float32)
    for k, v in enumerate(vals):
        rec = jnp.where(li == k, v, rec)
    return rec


def _nms_body(cx_hbm, cy_hbm, w_hbm, h_hbm, s_hbm, out_hbm,
              x1, y1, x2, y2, area, sc, recbuf, rec_all, outbuf, shared):
    wid = lax.axis_index("s")
    base = wid * PER_TILE

    pltpu.sync_copy(cx_hbm.at[pl.ds(base, PER_TILE)], x1)
    pltpu.sync_copy(w_hbm.at[pl.ds(base, PER_TILE)], x2)
    pltpu.sync_copy(cy_hbm.at[pl.ds(base, PER_TILE)], y1)
    pltpu.sync_copy(h_hbm.at[pl.ds(base, PER_TILE)], y2)
    pltpu.sync_copy(s_hbm.at[pl.ds(base, PER_TILE)], sc)

    lanes = _lanes_f32()

    def decode_chunk(c, carry):
        bv, bi = carry
        d = pl.ds(c * LANES, LANES)
        cxv = x1[d] * IMG_SIZE
        wv = x2[d] * IMG_SIZE
        cyv = y1[d] * IMG_SIZE
        hv = y2[d] * IMG_SIZE
        x1v = cxv - wv * 0.5
        x2v = cxv + wv * 0.5
        y1v = cyv - hv * 0.5
        y2v = cyv + hv * 0.5
        x1[d] = x1v
        x2[d] = x2v
        y1[d] = y1v
        y2[d] = y2v
        area[d] = jnp.maximum(x2v - x1v, 0.0) * jnp.maximum(y2v - y1v, 0.0)
        sv = sc[d]
        sv = jnp.where(sv > SCORE_THRESH, sv, NEG)
        sc[d] = sv
        idxv = (base + c * LANES).astype(jnp.float32) + lanes
        upd = sv > bv
        return jnp.where(upd, sv, bv), jnp.where(upd, idxv, bi)

    bv0 = jnp.full((LANES,), -BIG, jnp.float32)
    bi0 = jnp.zeros((LANES,), jnp.float32)
    bv, bi = lax.fori_loop(0, CHUNKS, decode_chunk, (bv0, bi0))

    def step(d, carry):
        bv, bi = carry
        lval = jnp.max(bv)
        lidx = jnp.min(jnp.where(bv == lval, bi, BIG))
        off = lidx.astype(jnp.int32) - base
        offv = jnp.zeros((LANES,), jnp.int32) + off
        gx1 = plsc.load_gather(x1, [offv])
        gy1 = plsc.load_gather(y1, [offv])
        gx2 = plsc.load_gather(x2, [offv])
        gy2 = plsc.load_gather(y2, [offv])
        gar = plsc.load_gather(area, [offv])
        recbuf[...] = _build_record([lval, lidx, gx1, gy1, gx2, gy2, gar])
        pbase = (d % 2) * (NUM_TILES * LANES)
        pltpu.sync_copy(recbuf, shared.at[pl.ds(pbase + wid * LANES, LANES)])
        plsc.subcore_barrier()
        pltpu.sync_copy(shared.at[pl.ds(pbase, NUM_TILES * LANES)], rec_all)

        li = lax.iota(jnp.int32, LANES)

        vals = plsc.load_gather(rec_all, [li * LANES])
        wv_ = jnp.max(vals)
        wtf = jnp.min(jnp.where(vals == wv_, lanes, BIG))
        rbase = wtf.astype(jnp.int32) * LANES
        rbv = jnp.zeros((LANES,), jnp.int32) + rbase
        wrec = rec_all[pl.ds(rbase, LANES)]
        wi_ = plsc.load_gather(rec_all, [rbv + 1])
        wx1 = plsc.load_gather(rec_all, [rbv + 2])
        wy1 = plsc.load_gather(rec_all, [rbv + 3])
        wx2 = plsc.load_gather(rec_all, [rbv + 4])
        wy2 = plsc.load_gather(rec_all, [rbv + 5])
        war = plsc.load_gather(rec_all, [rbv + 6])

        @pl.when(wid == 0)
        def _():
            valid = wv_ > 0.0
            z = jnp.float32(0.0)
            outbuf[d] = _build_record([
                jnp.where(valid, wx1, z),
                jnp.where(valid, wy1, z),
                jnp.where(valid, wx2, z),
                jnp.where(valid, wy2, z),
                jnp.where(valid, wv_, z),
            ])

        def sweep(c, carry):
            bv, bi = carry
            dd = pl.ds(c * LANES, LANES)
            x1v = x1[dd]
            y1v = y1[dd]
            x2v = x2[dd]
            y2v = y2[dd]
            ix1 = jnp.maximum(wx1, x1v)
            iy1 = jnp.maximum(wy1, y1v)
            ix2 = jnp.minimum(wx2, x2v)
            iy2 = jnp.minimum(wy2, y2v)
            inter = jnp.maximum(ix2 - ix1, 0.0) * jnp.maximum(iy2 - iy1, 0.0)
            union = war + area[dd] - inter
            iou = inter / jnp.maximum(union, 1e-9)
            idxv = (base + c * LANES).astype(jnp.float32) + lanes
            supp = (iou > IOU_THRESH) | (idxv == wi_)
            nv = jnp.where(supp, NEG, sc[dd])
            sc[dd] = nv
            upd = nv > bv
            return jnp.where(upd, nv, bv), jnp.where(upd, idxv, bi)

        bvn = jnp.full((LANES,), -BIG, jnp.float32)
        bin_ = jnp.zeros((LANES,), jnp.float32)
        return lax.fori_loop(0, CHUNKS, sweep, (bvn, bin_))

    lax.fori_loop(0, MAX_DET, step, (bv, bi))

    @pl.when(wid == 0)
    def _():
        pltpu.sync_copy(outbuf, out_hbm)


@jax.jit
def _nms_sc(cx, cy, w, h, s):
    mesh = plsc.VectorSubcoreMesh(
        core_axis_name="c", subcore_axis_name="s",
        num_cores=1, num_subcores=NUM_TILES)
    f = functools.partial(
        pl.kernel,
        out_type=jax.ShapeDtypeStruct((MAX_DET, LANES), jnp.float32),
        mesh=mesh,
        compiler_params=pltpu.CompilerParams(needs_layout_passes=False),
        scratch_types=[
            pltpu.VMEM((PER_TILE,), jnp.float32),
            pltpu.VMEM((PER_TILE,), jnp.float32),
            pltpu.VMEM((PER_TILE,), jnp.float32),
            pltpu.VMEM((PER_TILE,), jnp.float32),
            pltpu.VMEM((PER_TILE,), jnp.float32),
            pltpu.VMEM((PER_TILE,), jnp.float32),
            pltpu.VMEM((LANES,), jnp.float32),
            pltpu.VMEM((NUM_TILES * LANES,), jnp.float32),
            pltpu.VMEM((MAX_DET, LANES), jnp.float32),
            pltpu.VMEM_SHARED((2 * NUM_TILES * LANES,), jnp.float32),
        ],
    )(_nms_body)
    return f(cx, cy, w, h, s)


def kernel(boxes, scores):
    pad = PAD_N - N
    cx = jnp.pad(boxes[:, 0], (0, pad))
    cy = jnp.pad(boxes[:, 1], (0, pad))
    w = jnp.pad(boxes[:, 2], (0, pad))
    h = jnp.pad(boxes[:, 3], (0, pad))
    s = jnp.pad(scores, (0, pad))
    out = _nms_sc(cx, cy, w, h, s)
    return out[:, :5]
